# Initial kernel scaffold; baseline (speedup 1.0000x reference)
import jax
import jax.numpy as jnp
from jax.experimental import pallas as pl


def kernel(cat_tensor):
    raise NotImplementedError("write your pallas kernel here")


# speedup vs baseline: 2.0251x; 2.0251x over previous
import functools

import jax
import jax.numpy as jnp
from jax import lax
from jax.experimental import pallas as pl
from jax.experimental.pallas import tpu as pltpu
from jax.experimental.pallas import tpu_sc as plsc

B = 16384
F = 26
C = 100
OUT_D = F * C
NW = 32
ROWS_PER_W = B // NW
RBLK = 16
NBLK = ROWS_PER_W // RBLK
GRP = RBLK * F // 16
L = 16

_mesh = plsc.VectorSubcoreMesh(core_axis_name="c", subcore_axis_name="s")


@functools.partial(
    pl.kernel,
    out_type=jax.ShapeDtypeStruct((B, OUT_D), jnp.float32),
    mesh=_mesh,
    scratch_types=[
        pltpu.VMEM((ROWS_PER_W * F,), jnp.int32),
        pltpu.VMEM((RBLK, OUT_D), jnp.float32),
    ],
    compiler_params=pltpu.CompilerParams(needs_layout_passes=False),
)
def _one_hot_sc(cat_hbm, zeros_hbm, out_hbm, cat_v, buf_v):
    cid = lax.axis_index("c")
    sid = lax.axis_index("s")
    wid = sid * 2 + cid
    base = wid * ROWS_PER_W

    pltpu.sync_copy(cat_hbm.at[pl.ds(base * F, ROWS_PER_W * F)], cat_v)
    pltpu.sync_copy(zeros_hbm, buf_v)

    ones = jnp.ones((L,), jnp.float32)
    zeros = jnp.zeros((L,), jnp.float32)
    iota = lax.iota(jnp.int32, L)

    def scatter_pass(bi, vals):
        boff = bi * (RBLK * F)
        for g in range(GRP):
            p = iota + g * L
            r = p // F
            fld = p - r * F
            v = cat_v[pl.ds(boff + g * L, L)]
            v = jnp.minimum(jnp.maximum(v, 0), C - 1)
            plsc.store_scatter(buf_v, [r, fld * C + v], vals)

    @pl.loop(0, NBLK)
    def block(bi):
        scatter_pass(bi, ones)
        pltpu.sync_copy(buf_v, out_hbm.at[pl.ds(base + bi * RBLK, RBLK)])
        scatter_pass(bi, zeros)


@jax.jit
def _run(cat_tensor):
    flat = cat_tensor.reshape(-1).astype(jnp.int32)
    zeros = jnp.zeros((RBLK, OUT_D), jnp.float32)
    return _one_hot_sc(flat, zeros)


def kernel(cat_tensor):
    if cat_tensor.ndim == 1:
        cat_tensor = cat_tensor[None, :]
    return _run(cat_tensor)


# speedup vs baseline: 2.0552x; 1.0149x over previous
import functools

import jax
import jax.numpy as jnp
from jax import lax
from jax.experimental import pallas as pl
from jax.experimental.pallas import tpu as pltpu
from jax.experimental.pallas import tpu_sc as plsc

B = 16384
F = 26
C = 100
OUT_D = F * C
NW = 32
ROWS_PER_W = B // NW
RBLK = 16
NBLK = ROWS_PER_W // RBLK
GRP = RBLK * F // 16
L = 16

_mesh = plsc.VectorSubcoreMesh(core_axis_name="c", subcore_axis_name="s")


@functools.partial(
    pl.kernel,
    out_type=jax.ShapeDtypeStruct((B, OUT_D), jnp.float32),
    mesh=_mesh,
    scratch_types=[
        pltpu.VMEM((ROWS_PER_W * F,), jnp.int32),
        pltpu.VMEM((RBLK, OUT_D), jnp.float32),
        pltpu.VMEM((RBLK, OUT_D), jnp.float32),
        pltpu.SemaphoreType.DMA,
        pltpu.SemaphoreType.DMA,
    ],
    compiler_params=pltpu.CompilerParams(needs_layout_passes=False),
)
def _one_hot_sc(cat_hbm, zeros_hbm, out_hbm, cat_v, buf0, buf1, sem0, sem1):
    cid = lax.axis_index("c")
    sid = lax.axis_index("s")
    wid = sid * 2 + cid
    base = wid * ROWS_PER_W
    bufs = (buf0, buf1)
    sems = (sem0, sem1)

    pltpu.sync_copy(cat_hbm.at[pl.ds(base * F, ROWS_PER_W * F)], cat_v)
    pltpu.sync_copy(zeros_hbm, buf0)
    pltpu.sync_copy(zeros_hbm, buf1)

    ones = jnp.ones((L,), jnp.float32)
    zeros = jnp.zeros((L,), jnp.float32)
    iota = lax.iota(jnp.int32, L)

    def scatter_pass(bi, buf, vals):
        boff = bi * (RBLK * F)
        for g in range(GRP):
            p = iota + g * L
            r = p // F
            fld = p - r * F
            v = cat_v[pl.ds(boff + g * L, L)]
            v = jnp.minimum(jnp.maximum(v, 0), C - 1)
            plsc.store_scatter(buf, [r, fld * C + v], vals)

    def out_slice(bi):
        return out_hbm.at[pl.ds(base + bi * RBLK, RBLK)]

    for k in range(2):
        scatter_pass(k, bufs[k], ones)
        pltpu.make_async_copy(bufs[k], out_slice(k), sems[k]).start()

    @pl.loop(2, NBLK, step=2)
    def block_pair(m):
        for k in range(2):
            bi = m + k
            pltpu.make_async_copy(bufs[k], out_slice(bi), sems[k]).wait()
            scatter_pass(bi - 2, bufs[k], zeros)
            scatter_pass(bi, bufs[k], ones)
            pltpu.make_async_copy(bufs[k], out_slice(bi), sems[k]).start()

    for k in range(2):
        pltpu.make_async_copy(bufs[k], out_slice(k), sems[k]).wait()


@jax.jit
def _run(cat_tensor):
    flat = cat_tensor.reshape(-1).astype(jnp.int32)
    zeros = jnp.zeros((RBLK, OUT_D), jnp.float32)
    return _one_hot_sc(flat, zeros)


def kernel(cat_tensor):
    if cat_tensor.ndim == 1:
        cat_tensor = cat_tensor[None, :]
    return _run(cat_tensor)


# speedup vs baseline: 5.1173x; 2.4899x over previous
import functools

import jax
import jax.numpy as jnp
from jax import lax
from jax.experimental import pallas as pl
from jax.experimental.pallas import tpu as pltpu
from jax.experimental.pallas import tpu_sc as plsc

B = 16384
F = 26
C = 100
OUT_D = F * C
NW = 32
COLS_PER_W = B // NW
FPAIRS = F // 2
RROWS = 2 * C
CBLK = 256
NCL = COLS_PER_W // CBLK
GRP = CBLK // 16
L = 16

_mesh = plsc.VectorSubcoreMesh(core_axis_name="c", subcore_axis_name="s")


@functools.partial(
    pl.kernel,
    out_type=jax.ShapeDtypeStruct((OUT_D, B), jnp.float32),
    mesh=_mesh,
    scratch_types=[
        pltpu.VMEM((F * COLS_PER_W,), jnp.int32),
        pltpu.VMEM((RROWS, CBLK), jnp.float32),
        pltpu.VMEM((RROWS, CBLK), jnp.float32),
        pltpu.SemaphoreType.DMA,
        pltpu.SemaphoreType.DMA,
    ],
    compiler_params=pltpu.CompilerParams(needs_layout_passes=False),
)
def _one_hot_sc(cat_hbm, zeros_hbm, out_hbm, cat_v, buf0, buf1, sem0, sem1):
    cid = lax.axis_index("c")
    sid = lax.axis_index("s")
    wid = sid * 2 + cid
    cbase = wid * COLS_PER_W
    bufs = (buf0, buf1)
    sems = (sem0, sem1)

    for f in range(F):
        pltpu.sync_copy(
            cat_hbm.at[pl.ds(f * B + cbase, COLS_PER_W)],
            cat_v.at[pl.ds(f * COLS_PER_W, COLS_PER_W)],
        )
    pltpu.sync_copy(zeros_hbm, buf0)
    pltpu.sync_copy(zeros_hbm, buf1)

    ones = jnp.ones((L,), jnp.float32)
    zeros = jnp.zeros((L,), jnp.float32)
    iota = lax.iota(jnp.int32, L)

    def scatter_pass(fp, cl, buf, vals):
        for fl in range(2):
            voff = (fp * 2 + fl) * COLS_PER_W + cl * CBLK
            for g in range(GRP):
                v = cat_v[pl.ds(voff + g * L, L)]
                v = jnp.minimum(jnp.maximum(v, 0), C - 1)
                plsc.store_scatter(buf, [fl * C + v, iota + g * L], vals)

    def out_slice(fp, cl):
        return out_hbm.at[
            pl.ds(fp * RROWS, RROWS), pl.ds(cbase + cl * CBLK, CBLK)
        ]

    for cl in range(NCL):
        scatter_pass(0, cl, bufs[cl], ones)
        pltpu.make_async_copy(bufs[cl], out_slice(0, cl), sems[cl]).start()

    @pl.loop(1, FPAIRS)
    def fp_loop(fp):
        for cl in range(NCL):
            pltpu.make_async_copy(bufs[cl], out_slice(fp, cl), sems[cl]).wait()
            scatter_pass(fp - 1, cl, bufs[cl], zeros)
            scatter_pass(fp, cl, bufs[cl], ones)
            pltpu.make_async_copy(bufs[cl], out_slice(fp, cl), sems[cl]).start()

    for cl in range(NCL):
        pltpu.make_async_copy(bufs[cl], out_slice(0, cl), sems[cl]).wait()


@jax.jit
def _run(cat_tensor):
    flat_t = cat_tensor.T.reshape(-1).astype(jnp.int32)
    zeros = jnp.zeros((RROWS, CBLK), jnp.float32)
    return _one_hot_sc(flat_t, zeros).T


def kernel(cat_tensor):
    if cat_tensor.ndim == 1:
        cat_tensor = cat_tensor[None, :]
    return _run(cat_tensor)


# speedup vs baseline: 5.5447x; 1.0835x over previous
import functools

import jax
import jax.numpy as jnp
from jax import lax
from jax.experimental import pallas as pl
from jax.experimental.pallas import tpu as pltpu
from jax.experimental.pallas import tpu_sc as plsc

B = 16384
F = 26
C = 100
OUT_D = F * C
NW = 32
COLS_PER_W = B // NW
FPAIRS = F // 2
RROWS = 2 * C
CBLK = 256
NCL = COLS_PER_W // CBLK
GRP = CBLK // 16
L = 16

_mesh = plsc.VectorSubcoreMesh(core_axis_name="c", subcore_axis_name="s")


@functools.partial(
    pl.kernel,
    out_type=jax.ShapeDtypeStruct((OUT_D, B), jnp.float32),
    mesh=_mesh,
    scratch_types=[
        pltpu.VMEM((F, COLS_PER_W), jnp.int32),
        pltpu.VMEM((RROWS, CBLK), jnp.float32),
        pltpu.VMEM((RROWS, CBLK), jnp.float32),
        pltpu.SemaphoreType.DMA,
        pltpu.SemaphoreType.DMA,
    ],
    compiler_params=pltpu.CompilerParams(needs_layout_passes=False),
)
def _one_hot_sc(cat_hbm, zeros_hbm, out_hbm, cat_v, buf0, buf1, sem0, sem1):
    cid = lax.axis_index("c")
    sid = lax.axis_index("s")
    wid = sid * 2 + cid
    cbase = wid * COLS_PER_W
    bufs = (buf0, buf1)
    sems = (sem0, sem1)

    pltpu.sync_copy(cat_hbm.at[:, pl.ds(cbase, COLS_PER_W)], cat_v)
    pltpu.sync_copy(zeros_hbm, buf0)
    pltpu.sync_copy(zeros_hbm, buf1)

    ones = jnp.ones((L,), jnp.float32)
    zeros = jnp.zeros((L,), jnp.float32)
    iota = lax.iota(jnp.int32, L)

    def scatter_pass(fp, cl, buf, vals):
        for fl in range(2):
            fabs = fp * 2 + fl
            for g in range(GRP):
                v = cat_v[fabs, pl.ds(cl * CBLK + g * L, L)]
                v = jnp.minimum(jnp.maximum(v, 0), C - 1)
                plsc.store_scatter(buf, [fl * C + v, iota + g * L], vals)

    def out_slice(fp, cl):
        return out_hbm.at[
            pl.ds(fp * RROWS, RROWS), pl.ds(cbase + cl * CBLK, CBLK)
        ]

    for cl in range(NCL):
        scatter_pass(0, cl, bufs[cl], ones)
        pltpu.make_async_copy(bufs[cl], out_slice(0, cl), sems[cl]).start()

    @pl.loop(1, FPAIRS)
    def fp_loop(fp):
        for cl in range(NCL):
            pltpu.make_async_copy(bufs[cl], out_slice(fp, cl), sems[cl]).wait()
            scatter_pass(fp - 1, cl, bufs[cl], zeros)
            scatter_pass(fp, cl, bufs[cl], ones)
            pltpu.make_async_copy(bufs[cl], out_slice(fp, cl), sems[cl]).start()

    for cl in range(NCL):
        pltpu.make_async_copy(bufs[cl], out_slice(0, cl), sems[cl]).wait()


@jax.jit
def _run(cat_tensor):
    cat_t = cat_tensor.T.astype(jnp.int32)
    zeros = jnp.zeros((RROWS, CBLK), jnp.float32)
    return _one_hot_sc(cat_t, zeros).T


def kernel(cat_tensor):
    if cat_tensor.ndim == 1:
        cat_tensor = cat_tensor[None, :]
    return _run(cat_tensor)


# speedup vs baseline: 5.9088x; 1.0657x over previous
import functools

import jax
import jax.numpy as jnp
from jax import lax
from jax.experimental import pallas as pl
from jax.experimental.pallas import tpu as pltpu
from jax.experimental.pallas import tpu_sc as plsc

B = 16384
F = 26
C = 100
OUT_D = F * C
NW = 32
COLS_PER_W = B // NW
FPAIRS = F // 2
RROWS = 2 * C
CBLK = 256
NCL = COLS_PER_W // CBLK
GRP = CBLK // 16
L = 16

_mesh = plsc.VectorSubcoreMesh(core_axis_name="c", subcore_axis_name="s")


@functools.partial(
    pl.kernel,
    out_type=jax.ShapeDtypeStruct((OUT_D, B), jnp.float32),
    mesh=_mesh,
    scratch_types=[
        pltpu.VMEM((F, COLS_PER_W), jnp.int32),
        pltpu.VMEM((RROWS, CBLK), jnp.float32),
        pltpu.VMEM((RROWS, CBLK), jnp.float32),
        pltpu.SemaphoreType.DMA,
        pltpu.SemaphoreType.DMA,
    ],
    compiler_params=pltpu.CompilerParams(needs_layout_passes=False),
)
def _one_hot_sc(cat_hbm, zeros_hbm, out_hbm, cat_v, buf0, buf1, sem0, sem1):
    cid = lax.axis_index("c")
    sid = lax.axis_index("s")
    wid = sid * 2 + cid
    cbase = wid * COLS_PER_W
    bufs = (buf0, buf1)
    sems = (sem0, sem1)

    cat_cp = pltpu.make_async_copy(
        cat_hbm.at[:, pl.ds(cbase, COLS_PER_W)], cat_v, sem0
    )
    cat_cp.start()
    z0_cp = pltpu.make_async_copy(zeros_hbm, buf0, sem1)
    z0_cp.start()
    z1_cp = pltpu.make_async_copy(zeros_hbm.at[:], buf1, sem1)
    z1_cp.start()
    cat_cp.wait()
    z0_cp.wait()
    z1_cp.wait()

    ones = jnp.ones((L,), jnp.float32)
    zeros = jnp.zeros((L,), jnp.float32)
    iota = lax.iota(jnp.int32, L)

    def scatter_pass(fp, cl, buf, vals):
        for fl in range(2):
            fabs = fp * 2 + fl
            for g in range(GRP):
                v = cat_v[fabs, pl.ds(cl * CBLK + g * L, L)]
                v = jnp.minimum(jnp.maximum(v, 0), C - 1)
                plsc.store_scatter(buf, [fl * C + v, iota + g * L], vals)

    def out_slice(fp, cl):
        return out_hbm.at[
            pl.ds(fp * RROWS, RROWS), pl.ds(cbase + cl * CBLK, CBLK)
        ]

    scatter_pass(0, 0, buf0, ones)
    pltpu.make_async_copy(buf0, out_slice(0, 0), sem0).start()
    scatter_pass(0, 1, buf1, ones)
    pltpu.make_async_copy(buf1, out_slice(0, 1), sem1).start()

    @pl.loop(1, FPAIRS)
    def fp_loop(fp):
        for cl in range(NCL):
            pltpu.make_async_copy(bufs[cl], out_slice(fp, cl), sems[cl]).wait()
            scatter_pass(fp - 1, cl, bufs[cl], zeros)
            scatter_pass(fp, cl, bufs[cl], ones)
            pltpu.make_async_copy(bufs[cl], out_slice(fp, cl), sems[cl]).start()

    for cl in range(NCL):
        pltpu.make_async_copy(bufs[cl], out_slice(0, cl), sems[cl]).wait()


@jax.jit
def _run(cat_tensor):
    cat_t = cat_tensor.T.astype(jnp.int32)
    zeros = jnp.zeros((RROWS, CBLK), jnp.float32)
    return _one_hot_sc(cat_t, zeros).T


def kernel(cat_tensor):
    if cat_tensor.ndim == 1:
        cat_tensor = cat_tensor[None, :]
    return _run(cat_tensor)


# speedup vs baseline: 5.9327x; 1.0040x over previous
import functools

import jax
import jax.numpy as jnp
from jax import lax
from jax.experimental import pallas as pl
from jax.experimental.pallas import tpu as pltpu
from jax.experimental.pallas import tpu_sc as plsc

B = 16384
F = 26
C = 100
OUT_D = F * C
NW = 32
COLS_PER_W = B // NW
FPAIRS = F // 2
RROWS = 2 * C
CBLK = 256
NCL = COLS_PER_W // CBLK
GRP = CBLK // 16
L = 16

_mesh = plsc.VectorSubcoreMesh(core_axis_name="c", subcore_axis_name="s")


@functools.partial(
    pl.kernel,
    out_type=jax.ShapeDtypeStruct((OUT_D, B), jnp.float32),
    mesh=_mesh,
    scratch_types=[
        pltpu.VMEM((F, COLS_PER_W), jnp.int32),
        pltpu.VMEM((RROWS, CBLK), jnp.float32),
        pltpu.VMEM((RROWS, CBLK), jnp.float32),
        pltpu.SemaphoreType.DMA,
        pltpu.SemaphoreType.DMA,
    ],
    compiler_params=pltpu.CompilerParams(needs_layout_passes=False),
)
def _one_hot_sc(cat_hbm, zeros_hbm, out_hbm, cat_v, buf0, buf1, sem0, sem1):
    cid = lax.axis_index("c")
    sid = lax.axis_index("s")
    wid = cid * (NW // 2) + sid
    cbase = wid * COLS_PER_W
    bufs = (buf0, buf1)
    sems = (sem0, sem1)

    cat_cp = pltpu.make_async_copy(
        cat_hbm.at[:, pl.ds(cbase, COLS_PER_W)], cat_v, sem0
    )
    cat_cp.start()
    z0_cp = pltpu.make_async_copy(zeros_hbm, buf0, sem1)
    z0_cp.start()
    z1_cp = pltpu.make_async_copy(zeros_hbm.at[:], buf1, sem1)
    z1_cp.start()
    cat_cp.wait()
    z0_cp.wait()
    z1_cp.wait()

    ones = jnp.ones((L,), jnp.float32)
    zeros = jnp.zeros((L,), jnp.float32)
    iota = lax.iota(jnp.int32, L)

    def scatter_pass(fp, cl, buf, vals):
        for fl in range(2):
            fabs = fp * 2 + fl
            for g in range(GRP):
                v = cat_v[fabs, pl.ds(cl * CBLK + g * L, L)]
                v = jnp.minimum(jnp.maximum(v, 0), C - 1)
                plsc.store_scatter(buf, [fl * C + v, iota + g * L], vals)

    def out_slice(fp, cl):
        return out_hbm.at[
            pl.ds(fp * RROWS, RROWS), pl.ds(cbase + cl * CBLK, CBLK)
        ]

    scatter_pass(0, 0, buf0, ones)
    pltpu.make_async_copy(buf0, out_slice(0, 0), sem0).start()
    scatter_pass(0, 1, buf1, ones)
    pltpu.make_async_copy(buf1, out_slice(0, 1), sem1).start()

    @pl.loop(1, FPAIRS)
    def fp_loop(fp):
        for cl in range(NCL):
            pltpu.make_async_copy(bufs[cl], out_slice(fp, cl), sems[cl]).wait()
            scatter_pass(fp - 1, cl, bufs[cl], zeros)
            scatter_pass(fp, cl, bufs[cl], ones)
            pltpu.make_async_copy(bufs[cl], out_slice(fp, cl), sems[cl]).start()

    for cl in range(NCL):
        pltpu.make_async_copy(bufs[cl], out_slice(0, cl), sems[cl]).wait()


@jax.jit
def _run(cat_tensor):
    cat_t = cat_tensor.T.astype(jnp.int32)
    zeros = jnp.zeros((RROWS, CBLK), jnp.float32)
    return _one_hot_sc(cat_t, zeros).T


def kernel(cat_tensor):
    if cat_tensor.ndim == 1:
        cat_tensor = cat_tensor[None, :]
    return _run(cat_tensor)


# speedup vs baseline: 6.1508x; 1.0368x over previous
import functools

import jax
import jax.numpy as jnp
from jax import lax
from jax.experimental import pallas as pl
from jax.experimental.pallas import tpu as pltpu
from jax.experimental.pallas import tpu_sc as plsc

B = 16384
F = 26
C = 100
OUT_D = F * C
NW = 32
COLS_PER_W = B // NW
FPAIRS = F // 2
RROWS = 2 * C
CBLK = 256
NCL = COLS_PER_W // CBLK
GRP = CBLK // 16
L = 16

_mesh = plsc.VectorSubcoreMesh(core_axis_name="c", subcore_axis_name="s")


@functools.partial(
    pl.kernel,
    out_type=jax.ShapeDtypeStruct((OUT_D, B), jnp.float32),
    mesh=_mesh,
    scratch_types=[
        pltpu.VMEM((F, COLS_PER_W), jnp.int32),
        pltpu.VMEM((RROWS, CBLK), jnp.float32),
        pltpu.VMEM((RROWS, CBLK), jnp.float32),
        pltpu.SemaphoreType.DMA,
        pltpu.SemaphoreType.DMA,
    ],
    compiler_params=pltpu.CompilerParams(needs_layout_passes=False),
)
def _one_hot_sc(cat_hbm, zeros_hbm, out_hbm, cat_v, buf0, buf1, sem0, sem1):
    cid = lax.axis_index("c")
    sid = lax.axis_index("s")
    wid = cid * (NW // 2) + sid
    cbase = wid * COLS_PER_W
    bufs = (buf0, buf1)
    sems = (sem0, sem1)

    cat_cp = pltpu.make_async_copy(
        cat_hbm.at[:, pl.ds(cbase, COLS_PER_W)], cat_v, sem0
    )
    cat_cp.start()
    z0_cp = pltpu.make_async_copy(zeros_hbm, buf0, sem1)
    z0_cp.start()
    z1_cp = pltpu.make_async_copy(zeros_hbm.at[:], buf1, sem1)
    z1_cp.start()
    cat_cp.wait()
    z0_cp.wait()
    z1_cp.wait()

    ones = jnp.ones((L,), jnp.float32)
    zeros = jnp.zeros((L,), jnp.float32)
    iota = lax.iota(jnp.int32, L)

    def scatter_pass(fp, cl, buf, vals):
        for fl in range(2):
            @pl.loop(0, GRP)
            def g_loop(g):
                v = cat_v[fp * 2 + fl, pl.ds(cl * CBLK + g * L, L)]
                v = jnp.minimum(jnp.maximum(v, 0), C - 1)
                plsc.store_scatter(buf, [fl * C + v, iota + g * L], vals)

    def out_slice(fp, cl):
        return out_hbm.at[
            pl.ds(fp * RROWS, RROWS), pl.ds(cbase + cl * CBLK, CBLK)
        ]

    @pl.loop(0, FPAIRS)
    def fp_loop(fp):
        for cl in range(NCL):
            @pl.when(fp > 0)
            def _clear():
                pltpu.make_async_copy(
                    bufs[cl], out_slice(fp - 1, cl), sems[cl]
                ).wait()
                scatter_pass(fp - 1, cl, bufs[cl], zeros)

            scatter_pass(fp, cl, bufs[cl], ones)
            pltpu.make_async_copy(bufs[cl], out_slice(fp, cl), sems[cl]).start()

    for cl in range(NCL):
        pltpu.make_async_copy(bufs[cl], out_slice(0, cl), sems[cl]).wait()


@jax.jit
def _run(cat_tensor):
    cat_t = cat_tensor.T.astype(jnp.int32)
    zeros = jnp.zeros((RROWS, CBLK), jnp.float32)
    return _one_hot_sc(cat_t, zeros).T


def kernel(cat_tensor):
    if cat_tensor.ndim == 1:
        cat_tensor = cat_tensor[None, :]
    return _run(cat_tensor)
